# baseline (speedup 1.0000x reference)
import functools

import jax
import jax.numpy as jnp
from jax import lax
from jax.experimental import pallas as pl
from jax.experimental.pallas import tpu as pltpu
from jax.experimental.pallas import tpu_sc as plsc

N_NODES = 50000
B = 8192
K = 10
MSG = 64
MEM = 64
DEP = 2
PW = 2 * MSG

NC = 2
NS = 16
NW = NC * NS
BPW = B // NW
CH = 64
NCH = BPW // CH

_NB = B // 16


_LB = 25088
_NLB = 2


def _tc_pool_body(om, w1, b1, p_out):
    d = pl.program_id(1)
    k = pl.program_id(2)
    x = om[...]
    h = lax.dot_general(x, w1[0], (((0,), (1,)), ((), ())),
                        preferred_element_type=jnp.float32) + b1[0]
    h = jnp.maximum(h, 0.0)

    @pl.when(jnp.logical_and(d == 0, k == 0))
    def _init():
        p_out[...] = jnp.zeros((_LB, DEP * MSG), jnp.float32)

    @pl.when(d == 0)
    def _acc0():
        p_out[:, 0:MSG] = jnp.maximum(p_out[:, 0:MSG], h)

    @pl.when(d == 1)
    def _acc1():
        p_out[:, MSG:2 * MSG] = jnp.maximum(p_out[:, MSG:2 * MSG], h)


_tc_pool = pl.pallas_call(
    _tc_pool_body,
    grid=(_NLB, DEP, K),
    in_specs=[
        pl.BlockSpec((MSG, _LB), lambda l, d, k: (d * K + k, l)),
        pl.BlockSpec((1, MSG, MSG), lambda l, d, k: (d, 0, 0)),
        pl.BlockSpec((1, 1, MSG), lambda l, d, k: (d, 0, 0)),
    ],
    out_specs=pl.BlockSpec((_LB, DEP * MSG), lambda l, d, k: (l, 0)),
    out_shape=jax.ShapeDtypeStruct((N_NODES, DEP * MSG), jnp.float32),
)


@functools.cache
def _build_sc_winner():
    mesh = plsc.VectorSubcoreMesh(core_axis_name="c", subcore_axis_name="s")

    @functools.partial(
        pl.kernel,
        out_type=jax.ShapeDtypeStruct((B,), jnp.int32),
        mesh=mesh,
        compiler_params=pltpu.CompilerParams(needs_layout_passes=False),
        scratch_types=[
            pltpu.VMEM((N_NODES,), jnp.int32),
            pltpu.VMEM((B,), jnp.int32),
            pltpu.VMEM((B,), jnp.int32),
        ],
    )
    def _sc_winner(ids, w_out, pos_v, ids_v, w_v):
        wid = lax.axis_index("s") * NC + lax.axis_index("c")

        @pl.when(wid == 0)
        def _winner():
            pltpu.sync_copy(ids, ids_v)

            def pass1(jj, _):
                idxv = ids_v[pl.ds(jj * 16, 16)]
                bvec = lax.broadcasted_iota(jnp.int32, (16,), 0) + jj * 16
                plsc.store_scatter(pos_v, [idxv], bvec)
                return 0
            lax.fori_loop(0, _NB, pass1, 0, unroll=4)

            def fix_round(_):
                def body(jj, changed):
                    idxv = ids_v[pl.ds(jj * 16, 16)]
                    bvec = (lax.broadcasted_iota(jnp.int32, (16,), 0)
                            + jj * 16)
                    cur = plsc.load_gather(pos_v, [idxv])
                    m = bvec > cur
                    plsc.store_scatter(pos_v, [idxv], bvec, mask=m)
                    return changed | jnp.where(m, 1, 0)
                ch = lax.fori_loop(0, _NB, body, jnp.zeros((16,), jnp.int32))
                return jnp.max(ch)

            lax.while_loop(lambda c: c > 0, fix_round, jnp.int32(1))

            def final(jj, _):
                idxv = ids_v[pl.ds(jj * 16, 16)]
                w_v[pl.ds(jj * 16, 16)] = plsc.load_gather(pos_v, [idxv])
                return 0
            lax.fori_loop(0, _NB, final, 0, unroll=4)
            pltpu.sync_copy(w_v, w_out)

    return _sc_winner


@functools.cache
def _build_sc_gather():
    mesh = plsc.VectorSubcoreMesh(core_axis_name="c", subcore_axis_name="s")

    @functools.partial(
        pl.kernel,
        out_type=[
            jax.ShapeDtypeStruct((B, PW), jnp.float32),
            jax.ShapeDtypeStruct((B, PW), jnp.float32),
        ],
        mesh=mesh,
        compiler_params=pltpu.CompilerParams(needs_layout_passes=False),
        scratch_types=[
            pltpu.VMEM((CH,), jnp.int32),
            pltpu.VMEM((CH, PW), jnp.float32),
            pltpu.VMEM((CH, PW), jnp.float32),
            pltpu.SemaphoreType.DMA,
            pltpu.SemaphoreType.DMA,
        ],
    )
    def _sc_gather(p_tab, m_tab, ids, gp_out, gh_out,
                   idxc, bufp, bufh, s0, s1):
        wid = lax.axis_index("s") * NC + lax.axis_index("c")
        for c in range(NCH):
            base = pl.multiple_of(wid * BPW + c * CH, CH)
            pltpu.sync_copy(ids.at[pl.ds(base, CH)], idxc)
            cp0 = pltpu.async_copy(p_tab.at[idxc], bufp, s0)
            cp1 = pltpu.async_copy(m_tab.at[idxc], bufh, s1)
            cp0.wait()
            cp1.wait()
            pltpu.sync_copy(bufp, gp_out.at[pl.ds(base, CH)])
            pltpu.sync_copy(bufh, gh_out.at[pl.ds(base, CH)])

    return _sc_gather


_BB = 1024
_GRID = B // _BB


def _tc_dense_body(gp, gh, um, fc2t0, b20, fc2t1, b21, wih, bih, whh, bhh,
                   hn_out):
    pre = um[...]
    for d, (fc2t, b2) in enumerate(((fc2t0, b20), (fc2t1, b21))):
        pooled = gp[:, d * MSG:(d + 1) * MSG]
        cat = jnp.concatenate([pre, pooled], axis=1)
        pre = jnp.dot(cat, fc2t[...],
                      preferred_element_type=jnp.float32) + b2[...]
    h = gh[:, 0:MEM]
    gi = jnp.dot(pre, wih[...], preferred_element_type=jnp.float32) + bih[...]
    gh_ = jnp.dot(h, whh[...], preferred_element_type=jnp.float32) + bhh[...]
    r = jax.nn.sigmoid(gi[:, 0:MEM] + gh_[:, 0:MEM])
    z = jax.nn.sigmoid(gi[:, MEM:2 * MEM] + gh_[:, MEM:2 * MEM])
    n = jnp.tanh(gi[:, 2 * MEM:3 * MEM] + r * gh_[:, 2 * MEM:3 * MEM])
    hn = (1.0 - z) * n + z * h
    hn_out[...] = jnp.concatenate(
        [hn, jnp.zeros((_BB, PW - MEM), jnp.float32)], axis=1)


_tc_dense = pl.pallas_call(
    _tc_dense_body,
    grid=(_GRID,),
    in_specs=[
        pl.BlockSpec((_BB, PW), lambda i: (i, 0)),
        pl.BlockSpec((_BB, PW), lambda i: (i, 0)),
        pl.BlockSpec((_BB, MSG), lambda i: (i, 0)),
        pl.BlockSpec((2 * MSG, MSG), lambda i: (0, 0)),
        pl.BlockSpec((1, MSG), lambda i: (0, 0)),
        pl.BlockSpec((2 * MSG, MSG), lambda i: (0, 0)),
        pl.BlockSpec((1, MSG), lambda i: (0, 0)),
        pl.BlockSpec((MSG, 3 * MEM), lambda i: (0, 0)),
        pl.BlockSpec((1, 3 * MEM), lambda i: (0, 0)),
        pl.BlockSpec((MEM, 3 * MEM), lambda i: (0, 0)),
        pl.BlockSpec((1, 3 * MEM), lambda i: (0, 0)),
    ],
    out_specs=pl.BlockSpec((_BB, PW), lambda i: (i, 0)),
    out_shape=jax.ShapeDtypeStruct((B, PW), jnp.float32),
)


_SCH = 128
_SNCH = BPW // _SCH


@functools.cache
def _build_sc_scatter():
    mesh = plsc.VectorSubcoreMesh(core_axis_name="c", subcore_axis_name="s")

    @functools.partial(
        pl.kernel,
        out_type=(),
        mesh=mesh,
        scratch_types=[
            pltpu.VMEM((_SCH,), jnp.int32),
            pltpu.VMEM((_SCH,), jnp.int32),
            pltpu.VMEM((_SCH, PW), jnp.float32),
            pltpu.SemaphoreType.DMA,
        ],
    )
    def _sc_scatter(hn, wids, ids, out_ref, iv, wv, buf, s0):
        wid = lax.axis_index("s") * NC + lax.axis_index("c")
        for c in range(_SNCH):
            base = pl.multiple_of(wid * BPW + c * _SCH, _SCH)
            pltpu.sync_copy(wids.at[pl.ds(base, _SCH)], wv)
            pltpu.sync_copy(ids.at[pl.ds(base, _SCH)], iv)
            pltpu.async_copy(hn.at[wv], buf, s0).wait()
            pltpu.async_copy(buf, out_ref.at[iv], s0).wait()

    return _sc_scatter


def kernel(unique_messages, other_message, memory, fc1_w, fc1_b, fc2_w, fc2_b,
           W_ih, W_hh, b_ih, b_hh, unique_node_ids):
    ids = unique_node_ids
    om_t = other_message.transpose(0, 2, 3, 1).reshape(DEP * K * MSG, N_NODES)

    w = _build_sc_winner()(ids)
    p_node = _tc_pool(om_t, fc1_w, fc1_b.reshape(DEP, 1, MSG))
    mem_p = jnp.pad(memory, ((0, 0), (0, PW - MEM)))

    gp, gh = _build_sc_gather()(p_node, mem_p, ids)

    hn = _tc_dense(
        gp, gh, unique_messages,
        fc2_w[0].T, fc2_b[0].reshape(1, MSG),
        fc2_w[1].T, fc2_b[1].reshape(1, MSG),
        W_ih.T, b_ih.reshape(1, 3 * MEM),
        W_hh.T, b_hh.reshape(1, 3 * MEM),
    )

    out_ref = jax.new_ref(mem_p)
    _build_sc_scatter()(hn, w, ids, out_ref)
    return out_ref[...][:, :MEM]


# speedup vs baseline: 1.5340x; 1.5340x over previous
import functools

import jax
import jax.numpy as jnp
from jax import lax
from jax.experimental import pallas as pl
from jax.experimental.pallas import tpu as pltpu
from jax.experimental.pallas import tpu_sc as plsc

N_NODES = 50000
B = 8192
K = 10
MSG = 64
MEM = 64
DEP = 2
PW = 2 * MSG

NC = 2
NS = 16
NW = NC * NS
BPW = B // NW
CH = 64
NCH = BPW // CH

_NB = B // 16


_LB = 25088
_NLB = 2


def _tc_pool_body(om, w1, b1, p_out):
    k = pl.program_id(2)
    x = om[...]
    h = jnp.dot(w1[0], x, preferred_element_type=jnp.float32) + b1[0]
    h = jnp.maximum(h, 0.0)

    @pl.when(k == 0)
    def _init():
        p_out[...] = h[None]

    @pl.when(k > 0)
    def _acc():
        p_out[...] = jnp.maximum(p_out[...], h[None])


_tc_pool = pl.pallas_call(
    _tc_pool_body,
    grid=(DEP, _NLB, K),
    in_specs=[
        pl.BlockSpec((MSG, _LB), lambda d, l, k: (d * K + k, l)),
        pl.BlockSpec((1, MSG, MSG), lambda d, l, k: (d, 0, 0)),
        pl.BlockSpec((1, MSG, 1), lambda d, l, k: (d, 0, 0)),
    ],
    out_specs=pl.BlockSpec((1, MSG, _LB), lambda d, l, k: (d, 0, l)),
    out_shape=jax.ShapeDtypeStruct((DEP, MSG, N_NODES), jnp.float32),
)


@functools.cache
def _build_sc_winner():
    mesh = plsc.VectorSubcoreMesh(core_axis_name="c", subcore_axis_name="s")

    @functools.partial(
        pl.kernel,
        out_type=jax.ShapeDtypeStruct((B,), jnp.int32),
        mesh=mesh,
        compiler_params=pltpu.CompilerParams(needs_layout_passes=False),
        scratch_types=[
            pltpu.VMEM((N_NODES,), jnp.int32),
            pltpu.VMEM((B,), jnp.int32),
            pltpu.VMEM((B,), jnp.int32),
        ],
    )
    def _sc_winner(ids, w_out, pos_v, ids_v, w_v):
        wid = lax.axis_index("s") * NC + lax.axis_index("c")

        @pl.when(wid == 0)
        def _winner():
            pltpu.sync_copy(ids, ids_v)

            def pass1(jj, _):
                idxv = ids_v[pl.ds(jj * 16, 16)]
                bvec = lax.broadcasted_iota(jnp.int32, (16,), 0) + jj * 16
                plsc.store_scatter(pos_v, [idxv], bvec)
                return 0
            lax.fori_loop(0, _NB, pass1, 0, unroll=4)

            def fix_round(_):
                def body(jj, changed):
                    idxv = ids_v[pl.ds(jj * 16, 16)]
                    bvec = (lax.broadcasted_iota(jnp.int32, (16,), 0)
                            + jj * 16)
                    cur = plsc.load_gather(pos_v, [idxv])
                    m = bvec > cur
                    plsc.store_scatter(pos_v, [idxv], bvec, mask=m)
                    return changed | jnp.where(m, 1, 0)
                ch = lax.fori_loop(0, _NB, body, jnp.zeros((16,), jnp.int32))
                return jnp.max(ch)

            lax.while_loop(lambda c: c > 0, fix_round, jnp.int32(1))

            def final(jj, _):
                idxv = ids_v[pl.ds(jj * 16, 16)]
                w_v[pl.ds(jj * 16, 16)] = plsc.load_gather(pos_v, [idxv])
                return 0
            lax.fori_loop(0, _NB, final, 0, unroll=4)
            pltpu.sync_copy(w_v, w_out)

    return _sc_winner


@functools.cache
def _build_sc_gather():
    mesh = plsc.VectorSubcoreMesh(core_axis_name="c", subcore_axis_name="s")

    @functools.partial(
        pl.kernel,
        out_type=[
            jax.ShapeDtypeStruct((B, PW), jnp.float32),
            jax.ShapeDtypeStruct((B, PW), jnp.float32),
        ],
        mesh=mesh,
        compiler_params=pltpu.CompilerParams(needs_layout_passes=False),
        scratch_types=[
            pltpu.VMEM((CH,), jnp.int32),
            pltpu.VMEM((CH, PW), jnp.float32),
            pltpu.VMEM((CH, PW), jnp.float32),
            pltpu.SemaphoreType.DMA,
            pltpu.SemaphoreType.DMA,
        ],
    )
    def _sc_gather(p_tab, m_tab, ids, gp_out, gh_out,
                   idxc, bufp, bufh, s0, s1):
        wid = lax.axis_index("s") * NC + lax.axis_index("c")
        for c in range(NCH):
            base = pl.multiple_of(wid * BPW + c * CH, CH)
            pltpu.sync_copy(ids.at[pl.ds(base, CH)], idxc)
            cp0 = pltpu.async_copy(p_tab.at[idxc], bufp, s0)
            cp1 = pltpu.async_copy(m_tab.at[idxc], bufh, s1)
            cp0.wait()
            cp1.wait()
            pltpu.sync_copy(bufp, gp_out.at[pl.ds(base, CH)])
            pltpu.sync_copy(bufh, gh_out.at[pl.ds(base, CH)])

    return _sc_gather


_BB = 1024
_GRID = B // _BB


def _tc_dense_body(gp, gh, um, fc2t0, b20, fc2t1, b21, wih, bih, whh, bhh,
                   hn_out):
    pre = um[...]
    for d, (fc2t, b2) in enumerate(((fc2t0, b20), (fc2t1, b21))):
        pooled = gp[:, d * MSG:(d + 1) * MSG]
        cat = jnp.concatenate([pre, pooled], axis=1)
        pre = jnp.dot(cat, fc2t[...],
                      preferred_element_type=jnp.float32) + b2[...]
    h = gh[:, 0:MEM]
    gi = jnp.dot(pre, wih[...], preferred_element_type=jnp.float32) + bih[...]
    gh_ = jnp.dot(h, whh[...], preferred_element_type=jnp.float32) + bhh[...]
    r = jax.nn.sigmoid(gi[:, 0:MEM] + gh_[:, 0:MEM])
    z = jax.nn.sigmoid(gi[:, MEM:2 * MEM] + gh_[:, MEM:2 * MEM])
    n = jnp.tanh(gi[:, 2 * MEM:3 * MEM] + r * gh_[:, 2 * MEM:3 * MEM])
    hn = (1.0 - z) * n + z * h
    hn_out[...] = jnp.concatenate(
        [hn, jnp.zeros((_BB, PW - MEM), jnp.float32)], axis=1)


_tc_dense = pl.pallas_call(
    _tc_dense_body,
    grid=(_GRID,),
    in_specs=[
        pl.BlockSpec((_BB, PW), lambda i: (i, 0)),
        pl.BlockSpec((_BB, PW), lambda i: (i, 0)),
        pl.BlockSpec((_BB, MSG), lambda i: (i, 0)),
        pl.BlockSpec((2 * MSG, MSG), lambda i: (0, 0)),
        pl.BlockSpec((1, MSG), lambda i: (0, 0)),
        pl.BlockSpec((2 * MSG, MSG), lambda i: (0, 0)),
        pl.BlockSpec((1, MSG), lambda i: (0, 0)),
        pl.BlockSpec((MSG, 3 * MEM), lambda i: (0, 0)),
        pl.BlockSpec((1, 3 * MEM), lambda i: (0, 0)),
        pl.BlockSpec((MEM, 3 * MEM), lambda i: (0, 0)),
        pl.BlockSpec((1, 3 * MEM), lambda i: (0, 0)),
    ],
    out_specs=pl.BlockSpec((_BB, PW), lambda i: (i, 0)),
    out_shape=jax.ShapeDtypeStruct((B, PW), jnp.float32),
)


_SCH = 128
_SNCH = BPW // _SCH


@functools.cache
def _build_sc_scatter():
    mesh = plsc.VectorSubcoreMesh(core_axis_name="c", subcore_axis_name="s")

    @functools.partial(
        pl.kernel,
        out_type=(),
        mesh=mesh,
        scratch_types=[
            pltpu.VMEM((_SCH,), jnp.int32),
            pltpu.VMEM((_SCH,), jnp.int32),
            pltpu.VMEM((_SCH, PW), jnp.float32),
            pltpu.SemaphoreType.DMA,
        ],
    )
    def _sc_scatter(hn, wids, ids, out_ref, iv, wv, buf, s0):
        wid = lax.axis_index("s") * NC + lax.axis_index("c")
        for c in range(_SNCH):
            base = pl.multiple_of(wid * BPW + c * _SCH, _SCH)
            pltpu.sync_copy(wids.at[pl.ds(base, _SCH)], wv)
            pltpu.sync_copy(ids.at[pl.ds(base, _SCH)], iv)
            pltpu.async_copy(hn.at[wv], buf, s0).wait()
            pltpu.async_copy(buf, out_ref.at[iv], s0).wait()

    return _sc_scatter


def kernel(unique_messages, other_message, memory, fc1_w, fc1_b, fc2_w, fc2_b,
           W_ih, W_hh, b_ih, b_hh, unique_node_ids):
    ids = unique_node_ids
    om_t = other_message.transpose(0, 2, 3, 1).reshape(DEP * K * MSG, N_NODES)

    w = _build_sc_winner()(ids)
    p = _tc_pool(om_t, fc1_w, fc1_b.reshape(DEP, MSG, 1))
    p_node = p.reshape(DEP * MSG, N_NODES).T
    mem_p = jnp.pad(memory, ((0, 0), (0, PW - MEM)))

    gp, gh = _build_sc_gather()(p_node, mem_p, ids)

    hn = _tc_dense(
        gp, gh, unique_messages,
        fc2_w[0].T, fc2_b[0].reshape(1, MSG),
        fc2_w[1].T, fc2_b[1].reshape(1, MSG),
        W_ih.T, b_ih.reshape(1, 3 * MEM),
        W_hh.T, b_hh.reshape(1, 3 * MEM),
    )

    out_ref = jax.new_ref(mem_p)
    _build_sc_scatter()(hn, w, ids, out_ref)
    return out_ref[...][:, :MEM]


# speedup vs baseline: 1.6415x; 1.0700x over previous
import functools

import jax
import jax.numpy as jnp
from jax import lax
from jax.experimental import pallas as pl
from jax.experimental.pallas import tpu as pltpu
from jax.experimental.pallas import tpu_sc as plsc

N_NODES = 50000
B = 8192
K = 10
MSG = 64
MEM = 64
DEP = 2
PW = 2 * MSG

NC = 2
NS = 16
NW = NC * NS
BPW = B // NW
CH = 64
NCH = BPW // CH

_NB = B // 16


_LB = 25088
_NLB = 2


def _tc_pool_body(om, w1, b1, p_out):
    k = pl.program_id(2)
    x = om[...]
    h = jnp.dot(w1[0], x, preferred_element_type=jnp.float32) + b1[0]
    h = jnp.maximum(h, 0.0)

    @pl.when(k == 0)
    def _init():
        p_out[...] = h[None]

    @pl.when(k > 0)
    def _acc():
        p_out[...] = jnp.maximum(p_out[...], h[None])


_tc_pool = pl.pallas_call(
    _tc_pool_body,
    grid=(DEP, _NLB, K),
    in_specs=[
        pl.BlockSpec((MSG, _LB), lambda d, l, k: (d * K + k, l)),
        pl.BlockSpec((1, MSG, MSG), lambda d, l, k: (d, 0, 0)),
        pl.BlockSpec((1, MSG, 1), lambda d, l, k: (d, 0, 0)),
    ],
    out_specs=pl.BlockSpec((1, MSG, _LB), lambda d, l, k: (d, 0, l)),
    out_shape=jax.ShapeDtypeStruct((DEP, MSG, N_NODES), jnp.float32),
)


@functools.cache
def _build_sc_winner():
    mesh = plsc.VectorSubcoreMesh(core_axis_name="c", subcore_axis_name="s")

    @functools.partial(
        pl.kernel,
        out_type=jax.ShapeDtypeStruct((B,), jnp.int32),
        mesh=mesh,
        compiler_params=pltpu.CompilerParams(needs_layout_passes=False),
        scratch_types=[
            pltpu.VMEM((N_NODES,), jnp.int32),
            pltpu.VMEM((B,), jnp.int32),
            pltpu.VMEM((B,), jnp.int32),
        ],
    )
    def _sc_winner(ids, w_out, pos_v, ids_v, w_v):
        wid = lax.axis_index("s") * NC + lax.axis_index("c")

        @pl.when(wid == 0)
        def _winner():
            pltpu.sync_copy(ids, ids_v)

            def pass1(jj, _):
                idxv = ids_v[pl.ds(jj * 16, 16)]
                bvec = lax.broadcasted_iota(jnp.int32, (16,), 0) + jj * 16
                plsc.store_scatter(pos_v, [idxv], bvec)
                return 0
            lax.fori_loop(0, _NB, pass1, 0, unroll=4)

            def fix_round(_):
                def body(jj, changed):
                    idxv = ids_v[pl.ds(jj * 16, 16)]
                    bvec = (lax.broadcasted_iota(jnp.int32, (16,), 0)
                            + jj * 16)
                    cur = plsc.load_gather(pos_v, [idxv])
                    m = bvec > cur
                    plsc.store_scatter(pos_v, [idxv], bvec, mask=m)
                    return changed | jnp.where(m, 1, 0)
                ch = lax.fori_loop(0, _NB, body, jnp.zeros((16,), jnp.int32))
                return jnp.max(ch)

            lax.while_loop(lambda c: c > 0, fix_round, jnp.int32(1))

            def final(jj, _):
                idxv = ids_v[pl.ds(jj * 16, 16)]
                w_v[pl.ds(jj * 16, 16)] = plsc.load_gather(pos_v, [idxv])
                return 0
            lax.fori_loop(0, _NB, final, 0, unroll=4)
            pltpu.sync_copy(w_v, w_out)

    return _sc_winner


@functools.cache
def _build_sc_gather():
    mesh = plsc.VectorSubcoreMesh(core_axis_name="c", subcore_axis_name="s")

    @functools.partial(
        pl.kernel,
        out_type=[
            jax.ShapeDtypeStruct((B, PW), jnp.float32),
            jax.ShapeDtypeStruct((B, PW), jnp.float32),
        ],
        mesh=mesh,
        compiler_params=pltpu.CompilerParams(needs_layout_passes=False),
        scratch_types=[
            pltpu.VMEM((CH,), jnp.int32),
            pltpu.VMEM((CH, PW), jnp.float32),
            pltpu.VMEM((CH, PW), jnp.float32),
            pltpu.SemaphoreType.DMA,
            pltpu.SemaphoreType.DMA,
        ],
    )
    def _sc_gather(p_tab, m_tab, ids, gp_out, gh_out,
                   idxc, bufp, bufh, s0, s1):
        wid = lax.axis_index("s") * NC + lax.axis_index("c")
        for c in range(NCH):
            base = pl.multiple_of(wid * BPW + c * CH, CH)
            pltpu.sync_copy(ids.at[pl.ds(base, CH)], idxc)
            cp0 = pltpu.async_copy(p_tab.at[idxc], bufp, s0)
            cp1 = pltpu.async_copy(m_tab.at[idxc], bufh, s1)
            cp0.wait()
            cp1.wait()
            pltpu.sync_copy(bufp, gp_out.at[pl.ds(base, CH)])
            pltpu.sync_copy(bufh, gh_out.at[pl.ds(base, CH)])

    return _sc_gather


_BB = 1024
_GRID = B // _BB


def _tc_dense_body(gp, gh, um, fc2t0, b20, fc2t1, b21, wih, bih, whh, bhh,
                   hn_out):
    pre = um[...]
    for d, (fc2t, b2) in enumerate(((fc2t0, b20), (fc2t1, b21))):
        pooled = gp[:, d * MSG:(d + 1) * MSG]
        cat = jnp.concatenate([pre, pooled], axis=1)
        pre = jnp.dot(cat, fc2t[...],
                      preferred_element_type=jnp.float32) + b2[...]
    h = gh[:, 0:MEM]
    gi = jnp.dot(pre, wih[...], preferred_element_type=jnp.float32) + bih[...]
    gh_ = jnp.dot(h, whh[...], preferred_element_type=jnp.float32) + bhh[...]
    r = jax.nn.sigmoid(gi[:, 0:MEM] + gh_[:, 0:MEM])
    z = jax.nn.sigmoid(gi[:, MEM:2 * MEM] + gh_[:, MEM:2 * MEM])
    n = jnp.tanh(gi[:, 2 * MEM:3 * MEM] + r * gh_[:, 2 * MEM:3 * MEM])
    hn = (1.0 - z) * n + z * h
    hn_out[...] = jnp.concatenate(
        [hn, jnp.zeros((_BB, PW - MEM), jnp.float32)], axis=1)


_tc_dense = pl.pallas_call(
    _tc_dense_body,
    grid=(_GRID,),
    in_specs=[
        pl.BlockSpec((_BB, PW), lambda i: (i, 0)),
        pl.BlockSpec((_BB, PW), lambda i: (i, 0)),
        pl.BlockSpec((_BB, MSG), lambda i: (i, 0)),
        pl.BlockSpec((2 * MSG, MSG), lambda i: (0, 0)),
        pl.BlockSpec((1, MSG), lambda i: (0, 0)),
        pl.BlockSpec((2 * MSG, MSG), lambda i: (0, 0)),
        pl.BlockSpec((1, MSG), lambda i: (0, 0)),
        pl.BlockSpec((MSG, 3 * MEM), lambda i: (0, 0)),
        pl.BlockSpec((1, 3 * MEM), lambda i: (0, 0)),
        pl.BlockSpec((MEM, 3 * MEM), lambda i: (0, 0)),
        pl.BlockSpec((1, 3 * MEM), lambda i: (0, 0)),
    ],
    out_specs=pl.BlockSpec((_BB, PW), lambda i: (i, 0)),
    out_shape=jax.ShapeDtypeStruct((B, PW), jnp.float32),
)


_SCH = 128
_SNCH = BPW // _SCH


@functools.cache
def _build_sc_scatter():
    mesh = plsc.VectorSubcoreMesh(core_axis_name="c", subcore_axis_name="s")

    @functools.partial(
        pl.kernel,
        out_type=(),
        mesh=mesh,
        scratch_types=[
            pltpu.VMEM((_SCH,), jnp.int32),
            pltpu.VMEM((_SCH,), jnp.int32),
            pltpu.VMEM((_SCH, PW), jnp.float32),
            pltpu.SemaphoreType.DMA,
        ],
    )
    def _sc_scatter(hn, wids, ids, out_ref, iv, wv, buf, s0):
        wid = lax.axis_index("s") * NC + lax.axis_index("c")
        for c in range(_SNCH):
            base = pl.multiple_of(wid * BPW + c * _SCH, _SCH)
            pltpu.sync_copy(wids.at[pl.ds(base, _SCH)], wv)
            pltpu.sync_copy(ids.at[pl.ds(base, _SCH)], iv)
            pltpu.async_copy(hn.at[wv], buf, s0).wait()
            pltpu.async_copy(buf, out_ref.at[iv], s0).wait()

    return _sc_scatter


def kernel(unique_messages, other_message, memory, fc1_w, fc1_b, fc2_w, fc2_b,
           W_ih, W_hh, b_ih, b_hh, unique_node_ids):
    ids = unique_node_ids
    om_t = other_message.transpose(0, 2, 3, 1).reshape(DEP * K * MSG, N_NODES)

    w = _build_sc_winner()(ids)
    p = _tc_pool(om_t, fc1_w, fc1_b.reshape(DEP, MSG, 1))
    p, w = lax.optimization_barrier((p, w))
    p_node = p.reshape(DEP * MSG, N_NODES).T
    mem_p = jnp.pad(memory, ((0, 0), (0, PW - MEM)))

    gp, gh = _build_sc_gather()(p_node, mem_p, ids)

    hn = _tc_dense(
        gp, gh, unique_messages,
        fc2_w[0].T, fc2_b[0].reshape(1, MSG),
        fc2_w[1].T, fc2_b[1].reshape(1, MSG),
        W_ih.T, b_ih.reshape(1, 3 * MEM),
        W_hh.T, b_hh.reshape(1, 3 * MEM),
    )

    out_ref = jax.new_ref(mem_p)
    _build_sc_scatter()(hn, w, ids, out_ref)
    return out_ref[...][:, :MEM]
